# baseline (speedup 1.0000x reference)
import functools

import jax
import jax.numpy as jnp
from jax import lax
from jax.experimental import pallas as pl
from jax.experimental.pallas import tpu as pltpu
from jax.experimental.pallas import tpu_sc as plsc

_SCALE = 8.0


def _tc_transpose_table(tab_t, V, D):
    VB = 4096

    def body(in_ref, out_ref):
        t = jnp.transpose(in_ref[...], (1, 0))
        out_ref[...] = jnp.concatenate([t, t], axis=1)

    return pl.pallas_call(
        body,
        grid=((V + VB - 1) // VB,),
        in_specs=[pl.BlockSpec((D, VB), lambda v: (0, v))],
        out_specs=pl.BlockSpec((VB, 128), lambda v: (v, 0)),
        out_shape=jax.ShapeDtypeStruct((V, 128), jnp.float32),
    )(tab_t)


@functools.partial(jax.jit, static_argnums=(0, 1, 2))
def _sc_gather(B, V, D, idx_flat, tab_2d):
    info = plsc.get_sparse_core_info()
    NC, NS = info.num_cores, info.num_subcores
    NW = NC * NS
    b_per_w = B // NW
    CHUNK = 800
    n_chunks = b_per_w // CHUNK
    mesh = plsc.VectorSubcoreMesh(core_axis_name="c", subcore_axis_name="s")

    @functools.partial(
        pl.kernel,
        mesh=mesh,
        out_type=jax.ShapeDtypeStruct((B, 128), jnp.float32),
        scratch_types=[
            pltpu.VMEM((CHUNK,), jnp.int32),
            pltpu.VMEM((CHUNK,), jnp.int32),
            pltpu.VMEM((CHUNK, D), jnp.float32),
            pltpu.VMEM((CHUNK, D), jnp.float32),
            pltpu.SemaphoreType.DMA,
            pltpu.SemaphoreType.DMA,
            pltpu.SemaphoreType.DMA,
            pltpu.SemaphoreType.DMA,
        ],
        compiler_params=pltpu.CompilerParams(use_tc_tiling_on_sc=False),
    )
    def k(idx_hbm, tab_hbm, out_hbm, iv0, iv1, rv0, rv1, gs0, gs1, os0, os1):
        wid = lax.axis_index("s") * NC + lax.axis_index("c")
        base = wid * b_per_w
        iv = (iv0, iv1)
        rv = (rv0, rv1)
        gs = (gs0, gs1)
        osem = (os0, os1)

        def load_idx2(c, b):
            pltpu.sync_copy(
                idx_hbm.at[pl.ds(base + c * CHUNK, CHUNK)], iv[b])
            for u in range(CHUNK // 16):
                sl = pl.ds(u * 16, 16)
                iv[b][sl] = iv[b][sl] * 2

        load_idx2(0, 0)
        pltpu.async_copy(tab_hbm.at[iv0], rv0, gs0)
        load_idx2(1, 1)
        pltpu.async_copy(tab_hbm.at[iv1], rv1, gs1)

        @pl.loop(0, n_chunks - 2, step=2)
        def _steady(g):
            for b in range(2):
                c = g + b
                pltpu.make_async_copy(tab_hbm.at[iv[b]], rv[b], gs[b]).wait()
                oh = pltpu.async_copy(
                    rv[b],
                    out_hbm.at[pl.ds(base + c * CHUNK, CHUNK), pl.ds(0, D)],
                    osem[b])
                load_idx2(c + 2, b)
                oh.wait()
                pltpu.async_copy(tab_hbm.at[iv[b]], rv[b], gs[b])

        for c in (n_chunks - 2, n_chunks - 1):
            b = c % 2
            pltpu.make_async_copy(tab_hbm.at[iv[b]], rv[b], gs[b]).wait()
            pltpu.async_copy(
                rv[b],
                out_hbm.at[pl.ds(base + c * CHUNK, CHUNK), pl.ds(0, D)],
                osem[b]).wait()

    return k(idx_flat, tab_2d)


def _tc_finalize(gathered, BATCH, HIST, D):
    def body(in_ref, out_ref):
        out_ref[0] = jnp.transpose(in_ref[:, :D] * _SCALE, (1, 0))

    return pl.pallas_call(
        body,
        grid=(HIST,),
        in_specs=[pl.BlockSpec((BATCH, 128), lambda h: (h, 0))],
        out_specs=pl.BlockSpec((1, D, BATCH), lambda h: (h, 0, 0)),
        out_shape=jax.ShapeDtypeStruct((HIST, D, BATCH), jnp.float32),
    )(gathered)


def kernel(x, input_embedding):
    BATCH, HIST = x.shape
    V, D = input_embedding.shape
    B = BATCH * HIST
    idx = jnp.transpose(x).reshape(B)
    tab128 = _tc_transpose_table(jnp.transpose(input_embedding), V, D)
    gathered = _sc_gather(B, V, D, idx, tab128.reshape(2 * V, D))
    out_t = _tc_finalize(gathered, BATCH, HIST, D)
    return jnp.transpose(out_t, (2, 0, 1))


# speedup vs baseline: 1.1281x; 1.1281x over previous
import functools

import jax
import jax.numpy as jnp
from jax import lax
from jax.experimental import pallas as pl
from jax.experimental.pallas import tpu as pltpu
from jax.experimental.pallas import tpu_sc as plsc

_SCALE = 8.0
_SEGMENTS = 4


def _tc_transpose_table(tab_t, V, D):
    VB = 8192

    def body(in_ref, out_ref):
        t = jnp.transpose(in_ref[...], (1, 0))
        out_ref[...] = jnp.concatenate([t, t], axis=1)

    return pl.pallas_call(
        body,
        grid=((V + VB - 1) // VB,),
        in_specs=[pl.BlockSpec((D, VB), lambda v: (0, v))],
        out_specs=pl.BlockSpec((VB, 128), lambda v: (v, 0)),
        out_shape=jax.ShapeDtypeStruct((V, 128), jnp.float32),
    )(tab_t)


@functools.partial(jax.jit, static_argnums=(0, 1, 2, 3))
def _sc_gather_seg(B_seg, seg_base, V, D, idx_flat, tab_2d):
    info = plsc.get_sparse_core_info()
    NC, NS = info.num_cores, info.num_subcores
    NW = NC * NS
    b_per_w = B_seg // NW
    CHUNK = 800
    n_chunks = b_per_w // CHUNK
    mesh = plsc.VectorSubcoreMesh(core_axis_name="c", subcore_axis_name="s")

    @functools.partial(
        pl.kernel,
        mesh=mesh,
        out_type=jax.ShapeDtypeStruct((B_seg, 128), jnp.float32),
        scratch_types=[
            pltpu.VMEM((CHUNK,), jnp.int32),
            pltpu.VMEM((CHUNK,), jnp.int32),
            pltpu.VMEM((CHUNK, D), jnp.float32),
            pltpu.VMEM((CHUNK, D), jnp.float32),
            pltpu.SemaphoreType.DMA,
            pltpu.SemaphoreType.DMA,
            pltpu.SemaphoreType.DMA,
            pltpu.SemaphoreType.DMA,
        ],
        compiler_params=pltpu.CompilerParams(use_tc_tiling_on_sc=False),
    )
    def k(idx_hbm, tab_hbm, out_hbm, iv0, iv1, rv0, rv1, gs0, gs1, os0, os1):
        wid = lax.axis_index("s") * NC + lax.axis_index("c")
        base = seg_base + wid * b_per_w
        obase = wid * b_per_w
        iv = (iv0, iv1)
        rv = (rv0, rv1)
        gs = (gs0, gs1)
        osem = (os0, os1)

        def load_idx2(c, b):
            pltpu.sync_copy(
                idx_hbm.at[pl.ds(base + c * CHUNK, CHUNK)], iv[b])
            for u in range(CHUNK // 16):
                sl = pl.ds(u * 16, 16)
                iv[b][sl] = iv[b][sl] * 2

        load_idx2(0, 0)
        pltpu.async_copy(tab_hbm.at[iv0], rv0, gs0)
        load_idx2(1, 1)
        pltpu.async_copy(tab_hbm.at[iv1], rv1, gs1)

        @pl.loop(0, n_chunks - 2, step=2)
        def _steady(g):
            for b in range(2):
                c = g + b
                pltpu.make_async_copy(tab_hbm.at[iv[b]], rv[b], gs[b]).wait()
                oh = pltpu.async_copy(
                    rv[b],
                    out_hbm.at[pl.ds(obase + c * CHUNK, CHUNK), pl.ds(0, D)],
                    osem[b])
                load_idx2(c + 2, b)
                oh.wait()
                pltpu.async_copy(tab_hbm.at[iv[b]], rv[b], gs[b])

        for c in (n_chunks - 2, n_chunks - 1):
            b = c % 2
            pltpu.make_async_copy(tab_hbm.at[iv[b]], rv[b], gs[b]).wait()
            pltpu.async_copy(
                rv[b],
                out_hbm.at[pl.ds(obase + c * CHUNK, CHUNK), pl.ds(0, D)],
                osem[b]).wait()

    return k(idx_flat, tab_2d)


def _tc_finalize_seg(gathered, prev, h0, H_seg, BATCH, HIST, D):
    def body(in_ref, *rest):
        out_ref = rest[-1]
        out_ref[0] = jnp.transpose(in_ref[:, :D] * _SCALE, (1, 0))

    in_specs = [pl.BlockSpec((BATCH, 128), lambda h: (h, 0))]
    args = [gathered]
    aliases = {}
    if prev is not None:
        in_specs.append(pl.BlockSpec(memory_space=pl.ANY))
        args.append(prev)
        aliases = {1: 0}

    return pl.pallas_call(
        body,
        grid=(H_seg,),
        in_specs=in_specs,
        out_specs=pl.BlockSpec((1, D, BATCH), lambda h: (h0 + h, 0, 0)),
        out_shape=jax.ShapeDtypeStruct((HIST, D, BATCH), jnp.float32),
        input_output_aliases=aliases,
    )(*args)


def kernel(x, input_embedding):
    BATCH, HIST = x.shape
    V, D = input_embedding.shape
    B = BATCH * HIST
    S = _SEGMENTS
    H_seg = HIST // S
    B_seg = B // S
    idx = jnp.transpose(x).reshape(B)
    tab128 = _tc_transpose_table(jnp.transpose(input_embedding), V, D)
    tab2v = tab128.reshape(2 * V, D)
    gathered = [
        _sc_gather_seg(B_seg, s * B_seg, V, D, idx, tab2v) for s in range(S)
    ]
    out_t = _tc_finalize_seg(gathered[0], None, 0, H_seg, BATCH, HIST, D)
    for s in range(1, S):
        out_t = _tc_finalize_seg(
            gathered[s], out_t, s * H_seg, H_seg, BATCH, HIST, D)
    return jnp.transpose(out_t, (2, 0, 1))


# speedup vs baseline: 1.1312x; 1.0027x over previous
import functools

import jax
import jax.numpy as jnp
from jax import lax
from jax.experimental import pallas as pl
from jax.experimental.pallas import tpu as pltpu
from jax.experimental.pallas import tpu_sc as plsc

_SCALE = 8.0
_SEGMENTS = 8


def _tc_transpose_table(tab_t, V, D):
    VB = 8192

    def body(in_ref, out_ref):
        t = jnp.transpose(in_ref[...], (1, 0))
        out_ref[...] = jnp.concatenate([t, t], axis=1)

    return pl.pallas_call(
        body,
        grid=((V + VB - 1) // VB,),
        in_specs=[pl.BlockSpec((D, VB), lambda v: (0, v))],
        out_specs=pl.BlockSpec((VB, 128), lambda v: (v, 0)),
        out_shape=jax.ShapeDtypeStruct((V, 128), jnp.float32),
    )(tab_t)


@functools.partial(jax.jit, static_argnums=(0, 1, 2, 3))
def _sc_gather_seg(B_seg, seg_base, V, D, idx_flat, tab_2d):
    info = plsc.get_sparse_core_info()
    NC, NS = info.num_cores, info.num_subcores
    NW = NC * NS
    b_per_w = B_seg // NW
    CHUNK = 800
    n_chunks = b_per_w // CHUNK
    mesh = plsc.VectorSubcoreMesh(core_axis_name="c", subcore_axis_name="s")

    @functools.partial(
        pl.kernel,
        mesh=mesh,
        out_type=jax.ShapeDtypeStruct((B_seg, 128), jnp.float32),
        scratch_types=[
            pltpu.VMEM((CHUNK,), jnp.int32),
            pltpu.VMEM((CHUNK,), jnp.int32),
            pltpu.VMEM((CHUNK, D), jnp.float32),
            pltpu.VMEM((CHUNK, D), jnp.float32),
            pltpu.SemaphoreType.DMA,
            pltpu.SemaphoreType.DMA,
            pltpu.SemaphoreType.DMA,
            pltpu.SemaphoreType.DMA,
        ],
        compiler_params=pltpu.CompilerParams(use_tc_tiling_on_sc=False),
    )
    def k(idx_hbm, tab_hbm, out_hbm, iv0, iv1, rv0, rv1, gs0, gs1, os0, os1):
        wid = lax.axis_index("s") * NC + lax.axis_index("c")
        base = seg_base + wid * b_per_w
        obase = wid * b_per_w
        iv = (iv0, iv1)
        rv = (rv0, rv1)
        gs = (gs0, gs1)
        osem = (os0, os1)

        def load_idx2(c, b):
            pltpu.sync_copy(
                idx_hbm.at[pl.ds(base + c * CHUNK, CHUNK)], iv[b])
            for u in range(CHUNK // 16):
                sl = pl.ds(u * 16, 16)
                iv[b][sl] = iv[b][sl] * 2

        load_idx2(0, 0)
        pltpu.async_copy(tab_hbm.at[iv0], rv0, gs0)
        load_idx2(1, 1)
        pltpu.async_copy(tab_hbm.at[iv1], rv1, gs1)

        @pl.loop(0, n_chunks - 2, step=2)
        def _steady(g):
            for b in range(2):
                c = g + b
                pltpu.make_async_copy(tab_hbm.at[iv[b]], rv[b], gs[b]).wait()
                oh = pltpu.async_copy(
                    rv[b],
                    out_hbm.at[pl.ds(obase + c * CHUNK, CHUNK), pl.ds(0, D)],
                    osem[b])
                load_idx2(c + 2, b)
                oh.wait()
                pltpu.async_copy(tab_hbm.at[iv[b]], rv[b], gs[b])

        for c in (n_chunks - 2, n_chunks - 1):
            b = c % 2
            pltpu.make_async_copy(tab_hbm.at[iv[b]], rv[b], gs[b]).wait()
            pltpu.async_copy(
                rv[b],
                out_hbm.at[pl.ds(obase + c * CHUNK, CHUNK), pl.ds(0, D)],
                osem[b]).wait()

    return k(idx_flat, tab_2d)


def _tc_finalize_seg(gathered, prev, h0, H_seg, BATCH, HIST, D):
    def body(in_ref, *rest):
        out_ref = rest[-1]
        out_ref[0] = jnp.transpose(in_ref[:, :D] * _SCALE, (1, 0))

    in_specs = [pl.BlockSpec((BATCH, 128), lambda h: (h, 0))]
    args = [gathered]
    aliases = {}
    if prev is not None:
        in_specs.append(pl.BlockSpec(memory_space=pl.ANY))
        args.append(prev)
        aliases = {1: 0}

    return pl.pallas_call(
        body,
        grid=(H_seg,),
        in_specs=in_specs,
        out_specs=pl.BlockSpec((1, D, BATCH), lambda h: (h0 + h, 0, 0)),
        out_shape=jax.ShapeDtypeStruct((HIST, D, BATCH), jnp.float32),
        input_output_aliases=aliases,
    )(*args)


def kernel(x, input_embedding):
    BATCH, HIST = x.shape
    V, D = input_embedding.shape
    B = BATCH * HIST
    S = _SEGMENTS
    H_seg = HIST // S
    B_seg = B // S
    idx = jnp.transpose(x).reshape(B)
    tab128 = _tc_transpose_table(jnp.transpose(input_embedding), V, D)
    tab2v = tab128.reshape(2 * V, D)
    gathered = [
        _sc_gather_seg(B_seg, s * B_seg, V, D, idx, tab2v) for s in range(S)
    ]
    out_t = _tc_finalize_seg(gathered[0], None, 0, H_seg, BATCH, HIST, D)
    for s in range(1, S):
        out_t = _tc_finalize_seg(
            gathered[s], out_t, s * H_seg, H_seg, BATCH, HIST, D)
    return jnp.transpose(out_t, (2, 0, 1))


# speedup vs baseline: 1.1863x; 1.0487x over previous
import functools

import jax
import jax.numpy as jnp
from jax import lax
from jax.experimental import pallas as pl
from jax.experimental.pallas import tpu as pltpu
from jax.experimental.pallas import tpu_sc as plsc

_SCALE = 8.0
_SEGMENTS = 8


def _tc_transpose_table(tab_t, V, D):
    VB = 16384

    def body(in_ref, out_ref):
        t = jnp.transpose(in_ref[...], (1, 0))
        out_ref[...] = jnp.concatenate([t, t], axis=1)

    return pl.pallas_call(
        body,
        grid=((V + VB - 1) // VB,),
        in_specs=[pl.BlockSpec((D, VB), lambda v: (0, v))],
        out_specs=pl.BlockSpec((VB, 128), lambda v: (v, 0)),
        out_shape=jax.ShapeDtypeStruct((V, 128), jnp.float32),
    )(tab_t)


@functools.partial(jax.jit, static_argnums=(0, 1, 2, 3))
def _sc_gather_seg(B_seg, seg_base, V, D, idx_flat, tab_2d):
    info = plsc.get_sparse_core_info()
    NC, NS = info.num_cores, info.num_subcores
    NW = NC * NS
    b_per_w = B_seg // NW
    CHUNK = 800
    n_chunks = b_per_w // CHUNK
    mesh = plsc.VectorSubcoreMesh(core_axis_name="c", subcore_axis_name="s")

    @functools.partial(
        pl.kernel,
        mesh=mesh,
        out_type=jax.ShapeDtypeStruct((B_seg, 128), jnp.float32),
        scratch_types=[
            pltpu.VMEM((CHUNK,), jnp.int32),
            pltpu.VMEM((CHUNK,), jnp.int32),
            pltpu.VMEM((CHUNK, D), jnp.float32),
            pltpu.VMEM((CHUNK, D), jnp.float32),
            pltpu.SemaphoreType.DMA,
            pltpu.SemaphoreType.DMA,
            pltpu.SemaphoreType.DMA,
            pltpu.SemaphoreType.DMA,
        ],
        compiler_params=pltpu.CompilerParams(use_tc_tiling_on_sc=False),
    )
    def k(idx_hbm, tab_hbm, out_hbm, iv0, iv1, rv0, rv1, gs0, gs1, os0, os1):
        wid = lax.axis_index("s") * NC + lax.axis_index("c")
        base = seg_base + wid * b_per_w
        obase = wid * b_per_w
        iv = (iv0, iv1)
        rv = (rv0, rv1)
        gs = (gs0, gs1)
        osem = (os0, os1)

        def load_idx2(c, b):
            pltpu.sync_copy(
                idx_hbm.at[pl.ds(base + c * CHUNK, CHUNK)], iv[b])
            for u in range(CHUNK // 16):
                sl = pl.ds(u * 16, 16)
                iv[b][sl] = iv[b][sl] * 2

        load_idx2(0, 0)
        pltpu.async_copy(tab_hbm.at[iv0], rv0, gs0)
        load_idx2(1, 1)
        pltpu.async_copy(tab_hbm.at[iv1], rv1, gs1)

        @pl.loop(0, n_chunks - 2, step=2)
        def _steady(g):
            for b in range(2):
                c = g + b
                pltpu.make_async_copy(tab_hbm.at[iv[b]], rv[b], gs[b]).wait()
                oh = pltpu.async_copy(
                    rv[b],
                    out_hbm.at[pl.ds(obase + c * CHUNK, CHUNK), pl.ds(0, D)],
                    osem[b])
                load_idx2(c + 2, b)
                oh.wait()
                pltpu.async_copy(tab_hbm.at[iv[b]], rv[b], gs[b])

        for c in (n_chunks - 2, n_chunks - 1):
            b = c % 2
            pltpu.make_async_copy(tab_hbm.at[iv[b]], rv[b], gs[b]).wait()
            pltpu.async_copy(
                rv[b],
                out_hbm.at[pl.ds(obase + c * CHUNK, CHUNK), pl.ds(0, D)],
                osem[b]).wait()

    return k(idx_flat, tab_2d)


def _tc_finalize_seg(gathered, prev, h0, H_seg, BATCH, HIST, D):
    def body(in_ref, *rest):
        out_ref = rest[-1]
        out_ref[0] = jnp.transpose(in_ref[:, :D] * _SCALE, (1, 0))

    in_specs = [pl.BlockSpec((BATCH, 128), lambda h: (h, 0))]
    args = [gathered]
    aliases = {}
    if prev is not None:
        in_specs.append(pl.BlockSpec(memory_space=pl.ANY))
        args.append(prev)
        aliases = {1: 0}

    return pl.pallas_call(
        body,
        grid=(H_seg,),
        in_specs=in_specs,
        out_specs=pl.BlockSpec((1, D, BATCH), lambda h: (h0 + h, 0, 0)),
        out_shape=jax.ShapeDtypeStruct((HIST, D, BATCH), jnp.float32),
        input_output_aliases=aliases,
    )(*args)


def kernel(x, input_embedding):
    BATCH, HIST = x.shape
    V, D = input_embedding.shape
    B = BATCH * HIST
    S = _SEGMENTS
    H_seg = HIST // S
    B_seg = B // S
    idx = jnp.transpose(x).reshape(B)
    tab128 = _tc_transpose_table(jnp.transpose(input_embedding), V, D)
    tab2v = tab128.reshape(2 * V, D)
    gathered = [
        _sc_gather_seg(B_seg, s * B_seg, V, D, idx, tab2v) for s in range(S)
    ]
    out_t = _tc_finalize_seg(gathered[0], None, 0, H_seg, BATCH, HIST, D)
    for s in range(1, S):
        out_t = _tc_finalize_seg(
            gathered[s], out_t, s * H_seg, H_seg, BATCH, HIST, D)
    return jnp.transpose(out_t, (2, 0, 1))


# speedup vs baseline: 1.2011x; 1.0125x over previous
import functools

import jax
import jax.numpy as jnp
from jax import lax
from jax.experimental import pallas as pl
from jax.experimental.pallas import tpu as pltpu
from jax.experimental.pallas import tpu_sc as plsc

_SCALE = 8.0
_SEGMENTS = 8


def _tc_transpose_table(tab_t, V, D):
    VB = 25600

    def body(in_ref, out_ref):
        t = jnp.transpose(in_ref[...], (1, 0))
        out_ref[...] = jnp.concatenate([t, t], axis=1)

    return pl.pallas_call(
        body,
        grid=((V + VB - 1) // VB,),
        in_specs=[pl.BlockSpec((D, VB), lambda v: (0, v))],
        out_specs=pl.BlockSpec((VB, 128), lambda v: (v, 0)),
        out_shape=jax.ShapeDtypeStruct((V, 128), jnp.float32),
    )(tab_t)


@functools.partial(jax.jit, static_argnums=(0, 1, 2, 3))
def _sc_gather_seg(B_seg, seg_base, V, D, idx_flat, tab_2d):
    info = plsc.get_sparse_core_info()
    NC, NS = info.num_cores, info.num_subcores
    NW = NC * NS
    b_per_w = B_seg // NW
    CHUNK = 800
    n_chunks = b_per_w // CHUNK
    mesh = plsc.VectorSubcoreMesh(core_axis_name="c", subcore_axis_name="s")

    @functools.partial(
        pl.kernel,
        mesh=mesh,
        out_type=jax.ShapeDtypeStruct((B_seg, 128), jnp.float32),
        scratch_types=[
            pltpu.VMEM((CHUNK,), jnp.int32),
            pltpu.VMEM((CHUNK,), jnp.int32),
            pltpu.VMEM((CHUNK, D), jnp.float32),
            pltpu.VMEM((CHUNK, D), jnp.float32),
            pltpu.SemaphoreType.DMA,
            pltpu.SemaphoreType.DMA,
            pltpu.SemaphoreType.DMA,
            pltpu.SemaphoreType.DMA,
        ],
        compiler_params=pltpu.CompilerParams(use_tc_tiling_on_sc=False),
    )
    def k(idx_hbm, tab_hbm, out_hbm, iv0, iv1, rv0, rv1, gs0, gs1, os0, os1):
        wid = lax.axis_index("s") * NC + lax.axis_index("c")
        base = seg_base + wid * b_per_w
        obase = wid * b_per_w
        iv = (iv0, iv1)
        rv = (rv0, rv1)
        gs = (gs0, gs1)
        osem = (os0, os1)

        def load_idx2(c, b):
            pltpu.sync_copy(
                idx_hbm.at[pl.ds(base + c * CHUNK, CHUNK)], iv[b])
            for u in range(CHUNK // 16):
                sl = pl.ds(u * 16, 16)
                iv[b][sl] = iv[b][sl] * 2

        load_idx2(0, 0)
        pltpu.async_copy(tab_hbm.at[iv0], rv0, gs0)
        load_idx2(1, 1)
        pltpu.async_copy(tab_hbm.at[iv1], rv1, gs1)

        @pl.loop(0, n_chunks - 2, step=2)
        def _steady(g):
            for b in range(2):
                c = g + b
                pltpu.make_async_copy(tab_hbm.at[iv[b]], rv[b], gs[b]).wait()
                oh = pltpu.async_copy(
                    rv[b],
                    out_hbm.at[pl.ds(obase + c * CHUNK, CHUNK), pl.ds(0, D)],
                    osem[b])
                load_idx2(c + 2, b)
                oh.wait()
                pltpu.async_copy(tab_hbm.at[iv[b]], rv[b], gs[b])

        for c in (n_chunks - 2, n_chunks - 1):
            b = c % 2
            pltpu.make_async_copy(tab_hbm.at[iv[b]], rv[b], gs[b]).wait()
            pltpu.async_copy(
                rv[b],
                out_hbm.at[pl.ds(obase + c * CHUNK, CHUNK), pl.ds(0, D)],
                osem[b]).wait()

    return k(idx_flat, tab_2d)


def _tc_finalize_seg(gathered, prev, h0, H_seg, BATCH, HIST, D):
    def body(in_ref, *rest):
        out_ref = rest[-1]
        out_ref[0] = jnp.transpose(in_ref[:, :D] * _SCALE, (1, 0))

    in_specs = [pl.BlockSpec((BATCH, 128), lambda h: (h, 0))]
    args = [gathered]
    aliases = {}
    if prev is not None:
        in_specs.append(pl.BlockSpec(memory_space=pl.ANY))
        args.append(prev)
        aliases = {1: 0}

    return pl.pallas_call(
        body,
        grid=(H_seg,),
        in_specs=in_specs,
        out_specs=pl.BlockSpec((1, D, BATCH), lambda h: (h0 + h, 0, 0)),
        out_shape=jax.ShapeDtypeStruct((HIST, D, BATCH), jnp.float32),
        input_output_aliases=aliases,
    )(*args)


def kernel(x, input_embedding):
    BATCH, HIST = x.shape
    V, D = input_embedding.shape
    B = BATCH * HIST
    S = _SEGMENTS
    H_seg = HIST // S
    B_seg = B // S
    idx = jnp.transpose(x).reshape(B)
    tab128 = _tc_transpose_table(jnp.transpose(input_embedding), V, D)
    tab2v = tab128.reshape(2 * V, D)
    gathered = [
        _sc_gather_seg(B_seg, s * B_seg, V, D, idx, tab2v) for s in range(S)
    ]
    out_t = _tc_finalize_seg(gathered[0], None, 0, H_seg, BATCH, HIST, D)
    for s in range(1, S):
        out_t = _tc_finalize_seg(
            gathered[s], out_t, s * H_seg, H_seg, BATCH, HIST, D)
    return jnp.transpose(out_t, (2, 0, 1))


# speedup vs baseline: 1.3343x; 1.1109x over previous
import functools

import jax
import jax.numpy as jnp
from jax import lax
from jax.experimental import pallas as pl
from jax.experimental.pallas import tpu as pltpu
from jax.experimental.pallas import tpu_sc as plsc

_SCALE = 8.0
_SEGMENTS = 8


_VB2 = 16384


def _tc_transpose_table(tab_t, V, D):
    G = (V + 2 * _VB2 - 1) // (2 * _VB2)

    def body(a_ref, b_ref, out_ref):
        ta = jnp.transpose(a_ref[...], (1, 0))
        tb = jnp.transpose(b_ref[...], (1, 0))
        out_ref[...] = jnp.concatenate([ta, tb], axis=1)

    return pl.pallas_call(
        body,
        grid=(G,),
        in_specs=[
            pl.BlockSpec((D, _VB2), lambda j: (0, 2 * j)),
            pl.BlockSpec((D, _VB2), lambda j: (0, 2 * j + 1)),
        ],
        out_specs=pl.BlockSpec((_VB2, 128), lambda j: (j, 0)),
        out_shape=jax.ShapeDtypeStruct((G * _VB2, 128), jnp.float32),
    )(tab_t, tab_t)


@functools.partial(jax.jit, static_argnums=(0, 1, 2, 3))
def _sc_gather_seg(B_seg, seg_base, V, D, idx_flat, tab_2d):
    info = plsc.get_sparse_core_info()
    NC, NS = info.num_cores, info.num_subcores
    NW = NC * NS
    b_per_w = B_seg // NW
    CHUNK = 800
    n_chunks = b_per_w // CHUNK
    mesh = plsc.VectorSubcoreMesh(core_axis_name="c", subcore_axis_name="s")

    @functools.partial(
        pl.kernel,
        mesh=mesh,
        out_type=jax.ShapeDtypeStruct((B_seg, 128), jnp.float32),
        scratch_types=[
            pltpu.VMEM((CHUNK,), jnp.int32),
            pltpu.VMEM((CHUNK,), jnp.int32),
            pltpu.VMEM((CHUNK, D), jnp.float32),
            pltpu.VMEM((CHUNK, D), jnp.float32),
            pltpu.SemaphoreType.DMA,
            pltpu.SemaphoreType.DMA,
            pltpu.SemaphoreType.DMA,
            pltpu.SemaphoreType.DMA,
        ],
        compiler_params=pltpu.CompilerParams(use_tc_tiling_on_sc=False),
    )
    def k(idx_hbm, tab_hbm, out_hbm, iv0, iv1, rv0, rv1, gs0, gs1, os0, os1):
        wid = lax.axis_index("s") * NC + lax.axis_index("c")
        base = seg_base + wid * b_per_w
        obase = wid * b_per_w
        iv = (iv0, iv1)
        rv = (rv0, rv1)
        gs = (gs0, gs1)
        osem = (os0, os1)

        def load_idx2(c, b):
            pltpu.sync_copy(
                idx_hbm.at[pl.ds(base + c * CHUNK, CHUNK)], iv[b])
            for u in range(CHUNK // 16):
                sl = pl.ds(u * 16, 16)
                v = iv[b][sl]
                iv[b][sl] = ((v & (-2 * _VB2))
                             + ((v & (_VB2 - 1)) << 1)
                             + ((v >> 14) & 1))

        load_idx2(0, 0)
        pltpu.async_copy(tab_hbm.at[iv0], rv0, gs0)
        load_idx2(1, 1)
        pltpu.async_copy(tab_hbm.at[iv1], rv1, gs1)

        @pl.loop(0, n_chunks - 2, step=2)
        def _steady(g):
            for b in range(2):
                c = g + b
                pltpu.make_async_copy(tab_hbm.at[iv[b]], rv[b], gs[b]).wait()
                oh = pltpu.async_copy(
                    rv[b],
                    out_hbm.at[pl.ds(obase + c * CHUNK, CHUNK), pl.ds(0, D)],
                    osem[b])
                load_idx2(c + 2, b)
                oh.wait()
                pltpu.async_copy(tab_hbm.at[iv[b]], rv[b], gs[b])

        for c in (n_chunks - 2, n_chunks - 1):
            b = c % 2
            pltpu.make_async_copy(tab_hbm.at[iv[b]], rv[b], gs[b]).wait()
            pltpu.async_copy(
                rv[b],
                out_hbm.at[pl.ds(obase + c * CHUNK, CHUNK), pl.ds(0, D)],
                osem[b]).wait()

    return k(idx_flat, tab_2d)


def _tc_finalize_seg(gathered, prev, h0, H_seg, BATCH, HIST, D):
    def body(in_ref, *rest):
        out_ref = rest[-1]
        out_ref[0] = jnp.transpose(in_ref[:, :D] * _SCALE, (1, 0))

    in_specs = [pl.BlockSpec((BATCH, 128), lambda h: (h, 0))]
    args = [gathered]
    aliases = {}
    if prev is not None:
        in_specs.append(pl.BlockSpec(memory_space=pl.ANY))
        args.append(prev)
        aliases = {1: 0}

    return pl.pallas_call(
        body,
        grid=(H_seg,),
        in_specs=in_specs,
        out_specs=pl.BlockSpec((1, D, BATCH), lambda h: (h0 + h, 0, 0)),
        out_shape=jax.ShapeDtypeStruct((HIST, D, BATCH), jnp.float32),
        input_output_aliases=aliases,
    )(*args)


def kernel(x, input_embedding):
    BATCH, HIST = x.shape
    V, D = input_embedding.shape
    B = BATCH * HIST
    S = _SEGMENTS
    H_seg = HIST // S
    B_seg = B // S
    idx = jnp.transpose(x).reshape(B)
    tab128 = _tc_transpose_table(jnp.transpose(input_embedding), V, D)
    tab2v = tab128.reshape(2 * tab128.shape[0], D)
    gathered = [
        _sc_gather_seg(B_seg, s * B_seg, V, D, idx, tab2v) for s in range(S)
    ]
    out_t = _tc_finalize_seg(gathered[0], None, 0, H_seg, BATCH, HIST, D)
    for s in range(1, S):
        out_t = _tc_finalize_seg(
            gathered[s], out_t, s * H_seg, H_seg, BATCH, HIST, D)
    return jnp.transpose(out_t, (2, 0, 1))
